# Initial kernel scaffold; baseline (speedup 1.0000x reference)
import jax
import jax.numpy as jnp
from jax.experimental import pallas as pl


def kernel(anchors, rpn_cls_score, gt_boxes, gt_labels):
    raise NotImplementedError("write your pallas kernel here")


# speedup vs baseline: 27.1289x; 27.1289x over previous
import functools

import jax
import jax.numpy as jnp
from jax.experimental import pallas as pl

_POS_OV = 0.7
_NEG_OV = 0.3
_NUM_FG = 256
_NUM_BG = 256
_INT_MIN = -2147483648


def _assign_kernel(a_ref, score_ref, gt_ref, gl_ref, clst_ref, reg_ref,
                   clsw_ref, regw_ref, *, num_fg, num_bg):
  B, M, _ = gt_ref.shape
  K = a_ref.shape[1]
  f32 = jnp.float32

  ax1 = a_ref[0:1, :]
  ay1 = a_ref[1:2, :]
  ax2 = a_ref[2:3, :]
  ay2 = a_ref[3:4, :]
  area_a = jnp.maximum(ax2 - ax1, 0.0) * jnp.maximum(ay2 - ay1, 0.0)
  aw = jnp.maximum(ax2 - ax1, 1e-6)
  ah = jnp.maximum(ay2 - ay1, 1e-6)
  axc = ax1 + 0.5 * aw
  ayc = ay1 + 0.5 * ah
  midx = jax.lax.broadcasted_iota(jnp.int32, (M, K), 0)

  cls_t_rows = []
  cls_w_rows = []
  reg_w_rows = []
  for b in range(B):
    g = gt_ref[b]
    gl = gl_ref[b]
    gx1 = g[:, 0:1]
    gy1 = g[:, 1:2]
    gx2 = g[:, 2:3]
    gy2 = g[:, 3:4]
    x1 = jnp.maximum(ax1, gx1)
    y1 = jnp.maximum(ay1, gy1)
    x2 = jnp.minimum(ax2, gx2)
    y2 = jnp.minimum(ay2, gy2)
    inter = jnp.maximum(x2 - x1, 0.0) * jnp.maximum(y2 - y1, 0.0)
    area_g = jnp.maximum(gx2 - gx1, 0.0) * jnp.maximum(gy2 - gy1, 0.0)
    union = area_a + area_g - inter
    ov = inter / jnp.maximum(union, 1e-8)

    max_ov = jnp.max(ov, axis=0, keepdims=True)
    arg = jnp.min(jnp.where(ov == max_ov, midx, M), axis=0, keepdims=True)
    gt_max = jnp.max(ov, axis=1, keepdims=True)
    near_best = jnp.where(ov >= gt_max - 1e-5, 1.0, 0.0)
    is_best = (jnp.max(near_best, axis=0, keepdims=True) > 0.0) & (max_ov > 0.0)

    onehot = (midx == arg).astype(f32)
    glab = jnp.sum(onehot * gl, axis=0, keepdims=True)
    gx1g = jnp.sum(onehot * gx1, axis=0, keepdims=True)
    gy1g = jnp.sum(onehot * gy1, axis=0, keepdims=True)
    gx2g = jnp.sum(onehot * gx2, axis=0, keepdims=True)
    gy2g = jnp.sum(onehot * gy2, axis=0, keepdims=True)

    labels = jnp.where(max_ov < _NEG_OV, 0.0, -1.0)
    labels = jnp.where(is_best, 1.0, labels)
    labels = jnp.where(max_ov >= _POS_OV, 1.0, labels)
    cls_t = jnp.where(labels == 1.0, glab, labels)

    gw = jnp.maximum(gx2g - gx1g, 1e-6)
    gh = jnp.maximum(gy2g - gy1g, 1e-6)
    gxc = gx1g + 0.5 * gw
    gyc = gy1g + 0.5 * gh
    tx = (gxc - axc) / aw
    ty = (gyc - ayc) / ah
    tw = jnp.log(gw / aw)
    th = jnp.log(gh / ah)
    reg_ref[b] = jnp.concatenate([tx, ty, tw, th], axis=0)

    cls_w = (labels >= 0.0).astype(f32)
    reg_w = (labels == 1.0).astype(f32)
    clst_ref[b:b + 1, :] = cls_t
    cls_t_rows.append(cls_t)
    cls_w_rows.append(cls_w)
    reg_w_rows.append(reg_w)

  cls_t_all = jnp.concatenate(cls_t_rows, axis=0)
  cls_w_all = jnp.concatenate(cls_w_rows, axis=0)
  reg_w_all = jnp.concatenate(reg_w_rows, axis=0)
  score = score_ref[...]

  fg_elig = ((cls_t_all > 0.0) & (cls_w_all > 0.0)).astype(f32)
  bg_elig = ((cls_t_all == 0.0) & (cls_w_all > 0.0)).astype(f32)
  elig = jnp.concatenate([fg_elig, bg_elig], axis=0) > 0.0
  s_f = jnp.where(elig, jnp.concatenate([score, score], axis=0),
                  -jnp.inf)
  bits = jax.lax.bitcast_convert_type(s_f, jnp.int32)
  skey = jnp.where(bits >= 0, bits, bits ^ jnp.int32(0x7FFFFFFF))
  kvec = jnp.concatenate(
      [jnp.full((B, 1), num_fg, jnp.int32),
       jnp.full((B, 1), num_bg, jnp.int32)], axis=0)

  def bit_body(_, carry):
    prefix, bit = carry
    cand = prefix | bit
    cand_s = cand ^ _INT_MIN
    cnt = jnp.sum((skey >= cand_s).astype(jnp.int32), axis=1, keepdims=True)
    return jnp.where(cnt >= kvec, cand, prefix), jax.lax.shift_right_logical(
        bit, 1)

  prefix0 = jnp.zeros((2 * B, 1), jnp.int32)
  bit0 = jnp.full((2 * B, 1), _INT_MIN, jnp.int32)
  prefix, _ = jax.lax.fori_loop(0, 32, bit_body, (prefix0, bit0))
  thr = prefix ^ _INT_MIN

  n_gt = jnp.sum((skey > thr).astype(jnp.int32), axis=1, keepdims=True)
  need = kvec - n_gt
  tie = skey == thr
  posi = jax.lax.broadcasted_iota(jnp.int32, (2 * B, K), 1)

  def pos_body(_, carry):
    cut, bit = carry
    cand = cut | bit
    cnt = jnp.sum((tie & (posi < cand)).astype(jnp.int32), axis=1,
                  keepdims=True)
    return jnp.where(cnt <= need, cand, cut), jax.lax.shift_right_logical(
        bit, 1)

  cut0 = jnp.zeros((2 * B, 1), jnp.int32)
  bitp0 = jnp.full((2 * B, 1), jnp.int32(1 << 15))
  cut, _ = jax.lax.fori_loop(0, 16, pos_body, (cut0, bitp0))

  sel = elig & ((skey > thr) | (tie & (posi < cut)))
  mask = (sel[:B, :] | sel[B:, :]).astype(f32)
  clsw_ref[...] = cls_w_all * mask
  regw_ref[...] = reg_w_all * mask


@jax.jit
def kernel(anchors, rpn_cls_score, gt_boxes, gt_labels):
  K = anchors.shape[0]
  B, M, _ = gt_boxes.shape
  anchors_t = anchors.T
  gl_f = gt_labels.astype(jnp.float32)[..., None]
  body = functools.partial(_assign_kernel, num_fg=_NUM_FG, num_bg=_NUM_BG)
  cls_t, reg, cls_w, reg_w = pl.pallas_call(
      body,
      out_shape=(
          jax.ShapeDtypeStruct((B, K), jnp.float32),
          jax.ShapeDtypeStruct((B, 4, K), jnp.float32),
          jax.ShapeDtypeStruct((B, K), jnp.float32),
          jax.ShapeDtypeStruct((B, K), jnp.float32),
      ),
  )(anchors_t, rpn_cls_score, gt_boxes, gl_f)
  reg_t = jnp.transpose(reg, (0, 2, 1))
  return cls_t, reg_t, cls_w, reg_w
